# Initial kernel scaffold; baseline (speedup 1.0000x reference)
import jax
import jax.numpy as jnp
from jax.experimental import pallas as pl


def kernel(x, edge_index, W1_rel, W1_root, b1, W2_rel, W2_root, b2):
    raise NotImplementedError("write your pallas kernel here")


# speedup vs baseline: 19.2483x; 19.2483x over previous
import functools

import jax
import jax.numpy as jnp
from jax import lax
from jax.experimental import pallas as pl
from jax.experimental.pallas import tpu as pltpu
from jax.experimental.pallas import tpu_sc as plsc

D = 16
CHUNK = 128
NW = 32
BLK = 8


def _make_sc_scatter(n_acc, cpt):
    nblk = cpt // BLK
    zr = n_acc // 16
    mesh = plsc.VectorSubcoreMesh(core_axis_name="c", subcore_axis_name="s")

    @functools.partial(
        pl.kernel, mesh=mesh,
        out_type=jax.ShapeDtypeStruct((2, n_acc, D), jnp.float32),
        compiler_params=pltpu.CompilerParams(use_tc_tiling_on_sc=False),
        scratch_types=[
            pltpu.VMEM_SHARED((n_acc, D), jnp.float32),
            pltpu.VMEM((BLK, CHUNK), jnp.int32),
            pltpu.VMEM((BLK, CHUNK), jnp.int32),
            pltpu.VMEM((CHUNK, D), jnp.float32),
            pltpu.SemaphoreType.DMA,
        ],
    )
    def sc_scatter(x_hbm, src_hbm, dst_hbm, zeros_hbm, out_hbm,
                   acc, src_v, dst_v, rows_v, sem):
        c = lax.axis_index("c")
        s = lax.axis_index("s")
        wid = s * 2 + c
        pltpu.sync_copy(zeros_hbm.at[pl.ds(s * zr, zr)],
                        acc.at[pl.ds(s * zr, zr)])
        plsc.subcore_barrier()

        base = wid * cpt

        def body(i, carry):
            row0 = base + i * BLK
            pltpu.sync_copy(src_hbm.at[pl.ds(row0, BLK)], src_v)
            pltpu.sync_copy(dst_hbm.at[pl.ds(row0, BLK)], dst_v)
            for j in range(BLK):
                pltpu.async_copy(x_hbm.at[src_v.at[j]], rows_v, sem).wait()
                pltpu.sync_copy(rows_v, acc.at[dst_v.at[j]], add=True)
            return carry

        lax.fori_loop(0, nblk, body, 0)
        plsc.subcore_barrier()
        pltpu.sync_copy(acc.at[pl.ds(s * zr, zr)],
                        out_hbm.at[c, pl.ds(s * zr, zr)])

    return sc_scatter


def _dense(parts, x, wrT, wroT, b, rows_blk):
    n = x.shape[0]

    def body(p_ref, x_ref, wr_ref, wo_ref, b_ref, o_ref):
        p = p_ref[0] + p_ref[1]
        acc = jnp.dot(p, wr_ref[...], preferred_element_type=jnp.float32)
        acc += jnp.dot(x_ref[...], wo_ref[...], preferred_element_type=jnp.float32)
        o_ref[...] = jnp.maximum(acc + b_ref[...], 0.0)

    return pl.pallas_call(
        body,
        grid=(n // rows_blk,),
        in_specs=[
            pl.BlockSpec((2, rows_blk, D), lambda i: (0, i, 0)),
            pl.BlockSpec((rows_blk, D), lambda i: (i, 0)),
            pl.BlockSpec((D, D), lambda i: (0, 0)),
            pl.BlockSpec((D, D), lambda i: (0, 0)),
            pl.BlockSpec((1, D), lambda i: (0, 0)),
        ],
        out_specs=pl.BlockSpec((rows_blk, D), lambda i: (i, 0)),
        out_shape=jax.ShapeDtypeStruct((n, D), jnp.float32),
    )(parts, x, wrT, wroT, b)


def kernel(x, edge_index, W1_rel, W1_root, b1, W2_rel, W2_root, b2):
    n = x.shape[0]
    e = edge_index.shape[1]
    n_acc = -(-(n + 1) // CHUNK) * CHUNK

    cpt = -(-e // (NW * CHUNK))
    cpt = -(-cpt // BLK) * BLK
    e_pad = NW * cpt * CHUNK

    src = edge_index[0].astype(jnp.int32)
    dst = edge_index[1].astype(jnp.int32)
    pad = e_pad - e
    if pad:
        src = jnp.concatenate([src, jnp.zeros((pad,), jnp.int32)])
        dst = jnp.concatenate([dst, jnp.full((pad,), n, jnp.int32)])
    src2 = src.reshape(NW * cpt, CHUNK)
    dst2 = dst.reshape(NW * cpt, CHUNK)
    zeros = jnp.zeros((n_acc, D), jnp.float32)

    sc = _make_sc_scatter(n_acc, cpt)
    rows_blk = 4000

    p1 = sc(x, src2, dst2, zeros)
    h1 = _dense(p1, x, W1_rel.T, W1_root.T, b1.reshape(1, D), rows_blk)
    p2 = sc(h1, src2, dst2, zeros)
    h2 = _dense(p2, h1, W2_rel.T, W2_root.T, b2.reshape(1, D), rows_blk)
    return h2


# speedup vs baseline: 34.1143x; 1.7723x over previous
import functools

import jax
import jax.numpy as jnp
from jax import lax
from jax.experimental import pallas as pl
from jax.experimental.pallas import tpu as pltpu
from jax.experimental.pallas import tpu_sc as plsc

D = 16
CHUNK = 128
NW = 32
BLK = 4


def _make_sc_scatter(n_acc, cpt):
    nblk = cpt // BLK
    assert nblk % 2 == 0
    zr = n_acc // 16
    mesh = plsc.VectorSubcoreMesh(core_axis_name="c", subcore_axis_name="s")

    @functools.partial(
        pl.kernel, mesh=mesh,
        out_type=jax.ShapeDtypeStruct((2, n_acc, D), jnp.float32),
        compiler_params=pltpu.CompilerParams(use_tc_tiling_on_sc=False),
        scratch_types=[
            pltpu.VMEM_SHARED((n_acc, D), jnp.float32),
            pltpu.VMEM((BLK, CHUNK), jnp.int32),
            pltpu.VMEM((BLK, CHUNK), jnp.int32),
            pltpu.VMEM((BLK, CHUNK), jnp.int32),
            pltpu.VMEM((BLK, CHUNK), jnp.int32),
            pltpu.VMEM((BLK, CHUNK, D), jnp.float32),
            pltpu.VMEM((BLK, CHUNK, D), jnp.float32),
            pltpu.SemaphoreType.DMA,
            pltpu.SemaphoreType.DMA,
        ],
    )
    def sc_scatter(x_hbm, src_hbm, dst_hbm, zeros_hbm, out_hbm,
                   acc, srcA, dstA, srcB, dstB, rowsA, rowsB, semA, semB):
        c = lax.axis_index("c")
        s = lax.axis_index("s")
        wid = s * 2 + c
        pltpu.sync_copy(zeros_hbm.at[pl.ds(s * zr, zr)],
                        acc.at[pl.ds(s * zr, zr)])
        plsc.subcore_barrier()

        base = wid * cpt

        def load_idx(blk_row, src_v, dst_v):
            pltpu.sync_copy(src_hbm.at[pl.ds(blk_row, BLK)], src_v)
            pltpu.sync_copy(dst_hbm.at[pl.ds(blk_row, BLK)], dst_v)

        def fire(src_v, rows_v, sem):
            for j in range(BLK):
                pltpu.async_copy(x_hbm.at[src_v.at[j]], rows_v.at[j], sem)

        def drain(src_v, rows_v, sem):
            for j in range(BLK):
                pltpu.make_async_copy(x_hbm.at[src_v.at[j]], rows_v.at[j],
                                      sem).wait()

        def scat(dst_v, rows_v):
            for j in range(BLK):
                pltpu.sync_copy(rows_v.at[j], acc.at[dst_v.at[j]], add=True)

        load_idx(base, srcA, dstA)
        fire(srcA, rowsA, semA)

        def body(i, carry):
            row_a = base + (2 * i) * BLK
            load_idx(row_a + BLK, srcB, dstB)
            fire(srcB, rowsB, semB)
            drain(srcA, rowsA, semA)
            scat(dstA, rowsA)
            load_idx(row_a + 2 * BLK, srcA, dstA)
            fire(srcA, rowsA, semA)
            drain(srcB, rowsB, semB)
            scat(dstB, rowsB)
            return carry

        lax.fori_loop(0, nblk // 2, body, 0)
        drain(srcA, rowsA, semA)

        plsc.subcore_barrier()
        pltpu.sync_copy(acc.at[pl.ds(s * zr, zr)],
                        out_hbm.at[c, pl.ds(s * zr, zr)])

    return sc_scatter


def _dense(parts, x, wrT, wroT, b, rows_blk):
    n = x.shape[0]

    def body(p_ref, x_ref, wr_ref, wo_ref, b_ref, o_ref):
        p = p_ref[0] + p_ref[1]
        acc = jnp.dot(p, wr_ref[...], preferred_element_type=jnp.float32)
        acc += jnp.dot(x_ref[...], wo_ref[...], preferred_element_type=jnp.float32)
        o_ref[...] = jnp.maximum(acc + b_ref[...], 0.0)

    return pl.pallas_call(
        body,
        grid=(n // rows_blk,),
        in_specs=[
            pl.BlockSpec((2, rows_blk, D), lambda i: (0, i, 0)),
            pl.BlockSpec((rows_blk, D), lambda i: (i, 0)),
            pl.BlockSpec((D, D), lambda i: (0, 0)),
            pl.BlockSpec((D, D), lambda i: (0, 0)),
            pl.BlockSpec((1, D), lambda i: (0, 0)),
        ],
        out_specs=pl.BlockSpec((rows_blk, D), lambda i: (i, 0)),
        out_shape=jax.ShapeDtypeStruct((n, D), jnp.float32),
    )(parts, x, wrT, wroT, b)


def kernel(x, edge_index, W1_rel, W1_root, b1, W2_rel, W2_root, b2):
    n = x.shape[0]
    e = edge_index.shape[1]
    n_acc = -(-(n + 1) // CHUNK) * CHUNK

    cpt = -(-e // (NW * CHUNK))
    cpt = -(-cpt // (2 * BLK)) * 2 * BLK
    e_pad = NW * cpt * CHUNK

    src = edge_index[0].astype(jnp.int32)
    dst = edge_index[1].astype(jnp.int32)
    pad = e_pad - e
    if pad:
        src = jnp.concatenate([src, jnp.zeros((pad,), jnp.int32)])
        dst = jnp.concatenate([dst, jnp.full((pad,), n, jnp.int32)])
    src2 = src.reshape(NW * cpt, CHUNK)
    dst2 = dst.reshape(NW * cpt, CHUNK)
    tailpad = jnp.zeros((BLK, CHUNK), jnp.int32)
    src2 = jnp.concatenate([src2, tailpad])
    dst2 = jnp.concatenate([dst2, tailpad])
    zeros = jnp.zeros((n_acc, D), jnp.float32)

    sc = _make_sc_scatter(n_acc, cpt)
    rows_blk = 4000

    p1 = sc(x, src2, dst2, zeros)
    h1 = _dense(p1, x, W1_rel.T, W1_root.T, b1.reshape(1, D), rows_blk)
    p2 = sc(h1, src2, dst2, zeros)
    h2 = _dense(p2, h1, W2_rel.T, W2_root.T, b2.reshape(1, D), rows_blk)
    return h2
